# Initial kernel scaffold; baseline (speedup 1.0000x reference)
import functools

import jax
import jax.numpy as jnp
from jax import lax
from jax.experimental import pallas as pl
from jax.experimental.pallas import tpu as pltpu
from jax.experimental.pallas import tpu_sc as plsc

N = 10000
E = 320000
D = 128
DE = 16
LANES = 16

NT = 32
C = 128
NCH = 79
E_PAD = NT * NCH * C
NSUB = 16

AGG_ROWS = 10240
ZROWS = 640
ZCOPIES = ZROWS // C
DUMMY = 10200
OROWS = N // NSUB


def _ep_tc(ea, We, be):
    blk = 2048

    def body(ea_ref, we_ref, be_ref, out_ref):
        out_ref[...] = (
            jnp.dot(ea_ref[...], we_ref[...], preferred_element_type=jnp.float32)
            + be_ref[...]
        )

    return pl.pallas_call(
        body,
        grid=(E_PAD // blk,),
        in_specs=[
            pl.BlockSpec((blk, DE), lambda i: (i, 0)),
            pl.BlockSpec((DE, D), lambda i: (0, 0)),
            pl.BlockSpec((1, D), lambda i: (0, 0)),
        ],
        out_specs=pl.BlockSpec((blk, D), lambda i: (i, 0)),
        out_shape=jax.ShapeDtypeStruct((E_PAD, D), jnp.float32),
    )(ea, We, be.reshape(1, D))


def _update_tc(h, a0, a1, W, b, do_relu):
    def body(h_ref, a0_ref, a1_ref, w_ref, b_ref, out_ref):
        s = h_ref[...] + a0_ref[...] + a1_ref[...]
        y = jnp.dot(s, w_ref[...], preferred_element_type=jnp.float32) + b_ref[...]
        if do_relu:
            y = jnp.maximum(y, 0.0)
        out_ref[...] = y

    return pl.pallas_call(
        body,
        out_shape=jax.ShapeDtypeStruct((N, D), jnp.float32),
    )(h, a0, a1, W, b.reshape(1, D))


def _sc_layer(h, ep, src_p, dst_p):
    mesh = plsc.VectorSubcoreMesh(core_axis_name="c", subcore_axis_name="s")

    @functools.partial(
        pl.kernel,
        out_type=jax.ShapeDtypeStruct((2, N, D), jnp.float32),
        mesh=mesh,
        scratch_types=[
            pltpu.VMEM((NCH, C), jnp.int32),
            pltpu.VMEM((NCH, C), jnp.int32),
            pltpu.VMEM((C, D), jnp.float32),
            pltpu.VMEM((C, D), jnp.float32),
            pltpu.VMEM_SHARED((AGG_ROWS, D), jnp.float32),
            pltpu.SemaphoreType.DMA,
            pltpu.SemaphoreType.DMA,
        ],
    )
    def k(h_hbm, ep_hbm, src_hbm, dst_hbm, out_hbm,
          src_v, dst_v, hbuf, epbuf, agg, sem1, sem2):
        cid = lax.axis_index("c")
        sid = lax.axis_index("s")
        wid = cid * NSUB + sid

        @pl.loop(0, C)
        def _(i):
            for t in range(D // LANES):
                hbuf[i, pl.ds(t * LANES, LANES)] = jnp.zeros((LANES,), jnp.float32)

        for r in range(ZCOPIES):
            pltpu.sync_copy(hbuf, agg.at[pl.ds(sid * ZROWS + r * C, C)])
        plsc.subcore_barrier()

        pltpu.sync_copy(src_hbm.at[wid], src_v)
        pltpu.sync_copy(dst_hbm.at[wid], dst_v)

        @pl.loop(0, NCH)
        def _(j):
            cp1 = pltpu.async_copy(h_hbm.at[src_v.at[j]], hbuf, sem1)
            cp2 = pltpu.async_copy(ep_hbm.at[wid, j], epbuf, sem2)
            cp1.wait()
            cp2.wait()

            @pl.loop(0, C)
            def _(e):
                for t in range(D // LANES):
                    sl = pl.ds(t * LANES, LANES)
                    hbuf[e, sl] = jnp.maximum(hbuf[e, sl] + epbuf[e, sl], 0.0)

            pltpu.sync_copy(hbuf, agg.at[dst_v.at[j]], add=True)

        plsc.subcore_barrier()
        pltpu.sync_copy(agg.at[pl.ds(sid * OROWS, OROWS)],
                        out_hbm.at[cid, pl.ds(sid * OROWS, OROWS)])

    return k(h, ep, src_p, dst_p)


def kernel(x, edge_index, env_edge_attr, act_edge_attr,
           We0, be0, W0, b0, We1, be1, W1, b1, We2, be2, W2, b2):
    pad = E_PAD - E
    src = edge_index[0].astype(jnp.int32)
    dst = edge_index[1].astype(jnp.int32)
    src_p = jnp.concatenate([src, jnp.zeros((pad,), jnp.int32)]).reshape(NT, NCH, C)
    dst_p = jnp.concatenate([dst, jnp.full((pad,), DUMMY, jnp.int32)]).reshape(NT, NCH, C)
    zpad = jnp.zeros((pad, DE), jnp.float32)
    ea_env = jnp.concatenate([env_edge_attr, zpad])
    ea_act = jnp.concatenate([act_edge_attr, zpad])

    ep0 = _ep_tc(ea_env, We0, be0).reshape(NT, NCH, C, D)
    ep1 = _ep_tc(ea_act, We1, be1).reshape(NT, NCH, C, D)
    ep2 = _ep_tc(ea_act, We2, be2).reshape(NT, NCH, C, D)

    h = x
    for ep, W, b, relu_after in ((ep0, W0, b0, True),
                                 (ep1, W1, b1, True),
                                 (ep2, W2, b2, False)):
        aggp = _sc_layer(h, ep, src_p, dst_p)
        h = _update_tc(h, aggp[0], aggp[1], W, b, relu_after)
    return h


# speedup vs baseline: 2.6044x; 2.6044x over previous
import functools

import jax
import jax.numpy as jnp
from jax import lax
from jax.experimental import pallas as pl
from jax.experimental.pallas import tpu as pltpu
from jax.experimental.pallas import tpu_sc as plsc

N = 10000
E = 320000
D = 128
DE = 16
LANES = 16

NT = 32
NSUB = 16
C = 128
NCH = 79
NCH_PAD = 96
W_IDX = 32
NBLK = 3
E_PAD = NT * NCH * C

AGG_ROWS = 10112
ZROWS = 632
DUMMY = 10104


def _ep_tc(ea, We, be):
    blk = 2048

    def body(ea_ref, we_ref, be_ref, out_ref):
        out_ref[...] = (
            jnp.dot(ea_ref[...], we_ref[...], preferred_element_type=jnp.float32)
            + be_ref[...]
        )

    return pl.pallas_call(
        body,
        grid=(E_PAD // blk,),
        in_specs=[
            pl.BlockSpec((blk, DE), lambda i: (i, 0)),
            pl.BlockSpec((DE, D), lambda i: (0, 0)),
            pl.BlockSpec((1, D), lambda i: (0, 0)),
        ],
        out_specs=pl.BlockSpec((blk, D), lambda i: (i, 0)),
        out_shape=jax.ShapeDtypeStruct((E_PAD, D), jnp.float32),
    )(ea, We, be.reshape(1, D))


def _update_tc(h, a0, a1, W, b, do_relu):
    def body(h_ref, a0_ref, a1_ref, w_ref, b_ref, out_ref):
        s = h_ref[...] + a0_ref[...] + a1_ref[...]
        y = jnp.dot(s, w_ref[...], preferred_element_type=jnp.float32) + b_ref[...]
        if do_relu:
            y = jnp.maximum(y, 0.0)
        out_ref[...] = y

    return pl.pallas_call(
        body,
        out_shape=jax.ShapeDtypeStruct((N, D), jnp.float32),
    )(h, a0, a1, W, b.reshape(1, D))


def _sc_layer(h, ep, src_p, dst_p):
    mesh = plsc.VectorSubcoreMesh(core_axis_name="c", subcore_axis_name="s")

    @functools.partial(
        pl.kernel,
        out_type=jax.ShapeDtypeStruct((2, AGG_ROWS, D), jnp.float32),
        mesh=mesh,
        scratch_types=[
            pltpu.VMEM((W_IDX, C), jnp.int32),
            pltpu.VMEM((W_IDX, C), jnp.int32),
            pltpu.VMEM((C, D), jnp.float32),
            pltpu.VMEM((C, D), jnp.float32),
            pltpu.VMEM_SHARED((AGG_ROWS, D), jnp.float32),
            pltpu.SemaphoreType.DMA,
            pltpu.SemaphoreType.DMA,
        ],
    )
    def k(h_hbm, ep_hbm, src_hbm, dst_hbm, out_hbm,
          src_v, dst_v, hbuf, epbuf, agg, sem1, sem2):
        cid = lax.axis_index("c")
        sid = lax.axis_index("s")
        wid = cid * NSUB + sid

        @pl.loop(0, C)
        def _(i):
            for t in range(D // LANES):
                hbuf[i, pl.ds(t * LANES, LANES)] = jnp.zeros((LANES,), jnp.float32)

        for r in range(4):
            pltpu.sync_copy(hbuf, agg.at[pl.ds(sid * ZROWS + r * C, C)])
        pltpu.sync_copy(hbuf.at[pl.ds(0, ZROWS - 4 * C)],
                        agg.at[pl.ds(sid * ZROWS + 4 * C, ZROWS - 4 * C)])
        plsc.subcore_barrier()

        @pl.loop(0, NBLK)
        def _(blk):
            pltpu.sync_copy(src_hbm.at[wid, pl.ds(blk * W_IDX, W_IDX)], src_v)
            pltpu.sync_copy(dst_hbm.at[wid, pl.ds(blk * W_IDX, W_IDX)], dst_v)
            nin = lax.min(W_IDX, NCH - blk * W_IDX)

            @pl.loop(0, nin)
            def _(i):
                j = blk * W_IDX + i
                cp1 = pltpu.async_copy(h_hbm.at[src_v.at[i]], hbuf, sem1)
                cp2 = pltpu.async_copy(ep_hbm.at[wid, j], epbuf, sem2)
                cp1.wait()
                cp2.wait()

                @pl.loop(0, C)
                def _(e):
                    for t in range(D // LANES):
                        sl = pl.ds(t * LANES, LANES)
                        hbuf[e, sl] = jnp.maximum(hbuf[e, sl] + epbuf[e, sl], 0.0)

                pltpu.sync_copy(hbuf, agg.at[dst_v.at[i]], add=True)

        plsc.subcore_barrier()
        pltpu.sync_copy(agg.at[pl.ds(sid * ZROWS, ZROWS)],
                        out_hbm.at[cid, pl.ds(sid * ZROWS, ZROWS)])

    return k(h, ep, src_p, dst_p)


def kernel(x, edge_index, env_edge_attr, act_edge_attr,
           We0, be0, W0, b0, We1, be1, W1, b1, We2, be2, W2, b2):
    pad = E_PAD - E
    src = edge_index[0].astype(jnp.int32)
    dst = edge_index[1].astype(jnp.int32)
    src_p = jnp.concatenate([src, jnp.zeros((pad,), jnp.int32)]).reshape(NT, NCH, C)
    dst_p = jnp.concatenate([dst, jnp.full((pad,), DUMMY, jnp.int32)]).reshape(NT, NCH, C)
    src_p = jnp.pad(src_p, ((0, 0), (0, NCH_PAD - NCH), (0, 0)))
    dst_p = jnp.pad(dst_p, ((0, 0), (0, NCH_PAD - NCH), (0, 0)),
                    constant_values=DUMMY)
    zpad = jnp.zeros((pad, DE), jnp.float32)
    ea_env = jnp.concatenate([env_edge_attr, zpad])
    ea_act = jnp.concatenate([act_edge_attr, zpad])

    ep0 = _ep_tc(ea_env, We0, be0).reshape(NT, NCH, C, D)
    ep1 = _ep_tc(ea_act, We1, be1).reshape(NT, NCH, C, D)
    ep2 = _ep_tc(ea_act, We2, be2).reshape(NT, NCH, C, D)

    h = x
    for ep, W, b, relu_after in ((ep0, W0, b0, True),
                                 (ep1, W1, b1, True),
                                 (ep2, W2, b2, False)):
        aggp = _sc_layer(h, ep, src_p, dst_p)
        h = _update_tc(h, aggp[0, :N], aggp[1, :N], W, b, relu_after)
    return h


# speedup vs baseline: 2.8220x; 1.0836x over previous
import functools

import jax
import jax.numpy as jnp
from jax import lax
from jax.experimental import pallas as pl
from jax.experimental.pallas import tpu as pltpu
from jax.experimental.pallas import tpu_sc as plsc

N = 10000
E = 320000
D = 128
DE = 16
LANES = 16

NT = 32
NSUB = 16
C = 64
NCH = 158
NCH_PAD = 160
W_IDX = 32
NBLK = 5
E_PAD = NT * NCH * C

AGG_ROWS = 10112
ZROWS = 632
DUMMY = 10104


def _ep_tc(ea, We, be):
    blk = 2048

    def body(ea_ref, we_ref, be_ref, out_ref):
        out_ref[...] = (
            jnp.dot(ea_ref[...], we_ref[...], preferred_element_type=jnp.float32)
            + be_ref[...]
        )

    return pl.pallas_call(
        body,
        grid=(E_PAD // blk,),
        in_specs=[
            pl.BlockSpec((blk, DE), lambda i: (i, 0)),
            pl.BlockSpec((DE, D), lambda i: (0, 0)),
            pl.BlockSpec((1, D), lambda i: (0, 0)),
        ],
        out_specs=pl.BlockSpec((blk, D), lambda i: (i, 0)),
        out_shape=jax.ShapeDtypeStruct((E_PAD, D), jnp.float32),
    )(ea, We, be.reshape(1, D))


def _update_tc(h, aggp, W, b, do_relu):
    def body(h_ref, a_ref, w_ref, b_ref, out_ref):
        s = (h_ref[...]
             + a_ref[0, pl.ds(0, N), :]
             + a_ref[1, pl.ds(0, N), :])
        y = jnp.dot(s, w_ref[...], preferred_element_type=jnp.float32) + b_ref[...]
        if do_relu:
            y = jnp.maximum(y, 0.0)
        out_ref[...] = y

    return pl.pallas_call(
        body,
        out_shape=jax.ShapeDtypeStruct((N, D), jnp.float32),
    )(h, aggp, W, b.reshape(1, D))


def _sc_layer(h, ep, src_p, dst_p):
    mesh = plsc.VectorSubcoreMesh(core_axis_name="c", subcore_axis_name="s")

    @functools.partial(
        pl.kernel,
        out_type=jax.ShapeDtypeStruct((2, AGG_ROWS, D), jnp.float32),
        mesh=mesh,
        scratch_types=[
            pltpu.VMEM((W_IDX, C), jnp.int32),
            pltpu.VMEM((W_IDX, C), jnp.int32),
            pltpu.VMEM((C, D), jnp.float32),
            pltpu.VMEM((C, D), jnp.float32),
            pltpu.VMEM((C, D), jnp.float32),
            pltpu.VMEM((C, D), jnp.float32),
            pltpu.VMEM_SHARED((AGG_ROWS, D), jnp.float32),
            pltpu.SemaphoreType.DMA,
            pltpu.SemaphoreType.DMA,
            pltpu.SemaphoreType.DMA,
            pltpu.SemaphoreType.DMA,
        ],
    )
    def k(h_hbm, ep_hbm, src_hbm, dst_hbm, out_hbm,
          src_v, dst_v, hbuf0, hbuf1, epbuf0, epbuf1, agg,
          gsem0, gsem1, esem0, esem1):
        cid = lax.axis_index("c")
        sid = lax.axis_index("s")
        wid = cid * NSUB + sid

        def compute(hb, eb):
            @pl.loop(0, C)
            def _(e):
                for t in range(D // LANES):
                    sl = pl.ds(t * LANES, LANES)
                    hb[e, sl] = jnp.maximum(hb[e, sl] + eb[e, sl], 0.0)

        @pl.loop(0, C)
        def _(i):
            for t in range(D // LANES):
                hbuf0[i, pl.ds(t * LANES, LANES)] = jnp.zeros((LANES,), jnp.float32)

        for r in range(9):
            pltpu.sync_copy(hbuf0, agg.at[pl.ds(sid * ZROWS + r * C, C)])
        pltpu.sync_copy(hbuf0.at[pl.ds(0, ZROWS - 9 * C)],
                        agg.at[pl.ds(sid * ZROWS + 9 * C, ZROWS - 9 * C)])
        plsc.subcore_barrier()

        @pl.loop(0, NBLK)
        def _(blk):
            base = blk * W_IDX
            nin = lax.min(W_IDX, NCH - base)
            npair = nin // 2
            pltpu.sync_copy(src_hbm.at[wid, pl.ds(base, W_IDX)], src_v)
            pltpu.sync_copy(dst_hbm.at[wid, pl.ds(base, W_IDX)], dst_v)
            pltpu.async_copy(h_hbm.at[src_v.at[0]], hbuf0, gsem0)
            pltpu.async_copy(ep_hbm.at[wid, base], epbuf0, esem0)

            @pl.loop(0, npair)
            def _(p):
                a = 2 * p
                b = a + 1
                pltpu.make_async_copy(h_hbm.at[src_v.at[a]], hbuf0, gsem0).wait()
                pltpu.make_async_copy(ep_hbm.at[wid, base + a], epbuf0, esem0).wait()
                pltpu.async_copy(h_hbm.at[src_v.at[b]], hbuf1, gsem1)
                pltpu.async_copy(ep_hbm.at[wid, base + b], epbuf1, esem1)
                compute(hbuf0, epbuf0)
                pltpu.sync_copy(hbuf0, agg.at[dst_v.at[a]], add=True)
                pltpu.make_async_copy(h_hbm.at[src_v.at[b]], hbuf1, gsem1).wait()
                pltpu.make_async_copy(ep_hbm.at[wid, base + b], epbuf1, esem1).wait()

                @pl.when(a + 2 < nin)
                def _():
                    pltpu.async_copy(h_hbm.at[src_v.at[a + 2]], hbuf0, gsem0)
                    pltpu.async_copy(ep_hbm.at[wid, base + a + 2], epbuf0, esem0)

                compute(hbuf1, epbuf1)
                pltpu.sync_copy(hbuf1, agg.at[dst_v.at[b]], add=True)

        plsc.subcore_barrier()
        pltpu.sync_copy(agg.at[pl.ds(sid * ZROWS, ZROWS)],
                        out_hbm.at[cid, pl.ds(sid * ZROWS, ZROWS)])

    return k(h, ep, src_p, dst_p)


def kernel(x, edge_index, env_edge_attr, act_edge_attr,
           We0, be0, W0, b0, We1, be1, W1, b1, We2, be2, W2, b2):
    pad = E_PAD - E
    src = edge_index[0].astype(jnp.int32)
    dst = edge_index[1].astype(jnp.int32)
    src_p = jnp.concatenate([src, jnp.zeros((pad,), jnp.int32)]).reshape(NT, NCH, C)
    dst_p = jnp.concatenate([dst, jnp.full((pad,), DUMMY, jnp.int32)]).reshape(NT, NCH, C)
    src_p = jnp.pad(src_p, ((0, 0), (0, NCH_PAD - NCH), (0, 0)))
    dst_p = jnp.pad(dst_p, ((0, 0), (0, NCH_PAD - NCH), (0, 0)),
                    constant_values=DUMMY)
    zpad = jnp.zeros((pad, DE), jnp.float32)
    ea_env = jnp.concatenate([env_edge_attr, zpad])
    ea_act = jnp.concatenate([act_edge_attr, zpad])

    ep0 = _ep_tc(ea_env, We0, be0).reshape(NT, NCH, C, D)
    ep1 = _ep_tc(ea_act, We1, be1).reshape(NT, NCH, C, D)
    ep2 = _ep_tc(ea_act, We2, be2).reshape(NT, NCH, C, D)

    h = x
    for ep, W, b, relu_after in ((ep0, W0, b0, True),
                                 (ep1, W1, b1, True),
                                 (ep2, W2, b2, False)):
        aggp = _sc_layer(h, ep, src_p, dst_p)
        h = _update_tc(h, aggp, W, b, relu_after)
    return h
